# baseline (speedup 1.0000x reference)
import functools

import jax
import jax.numpy as jnp
from jax import lax
from jax.experimental import pallas as pl
from jax.experimental.pallas import tpu as pltpu
from jax.experimental.pallas import tpu_sc as plsc

VOCAB = 49408
MAXPOS = 8192
DIM = 1024
NSPECIAL = 16
LROWS = MAXPOS + NSPECIAL

NC = 2
NS = 16
LANES = 16
NW = NC * NS
RW = MAXPOS // NW
CHUNK = 32
NCHUNK = RW // CHUNK
VPR = DIM // LANES


def _sc_body(ids_hbm, tok_hbm, pos_hbm, spec_hbm, out_hbm,
             idx_all, tok_v, pos_v, stage_v,
             gsem0, gsem1, psem, wsem0, wsem1):
    gsems = (gsem0, gsem1)
    wsems = (wsem0, wsem1)
    wid = lax.axis_index("s") * NC + lax.axis_index("c")
    base = wid * RW

    pltpu.sync_copy(ids_hbm.at[pl.ds(NSPECIAL + base, RW)], idx_all)

    def gather(ch, b):
        return pltpu.async_copy(
            tok_hbm.at[idx_all.at[pl.ds(ch * CHUNK, CHUNK)]], tok_v.at[b],
            gsems[b])

    def issue_pos(ch):
        return pltpu.async_copy(pos_hbm.at[pl.ds(base + ch * CHUNK, CHUNK)],
                                pos_v, psem)

    inflight = [None, None]
    writes = [None, None]
    inflight[0] = gather(0, 0)
    pdesc = issue_pos(0)

    @pl.when(wid == 0)
    def _():
        pltpu.sync_copy(spec_hbm, stage_v)

    for ch in range(NCHUNK):
        b = ch & 1
        row0 = base + ch * CHUNK
        if ch + 1 < NCHUNK:
            if writes[1 - b] is not None:
                writes[1 - b].wait()
                writes[1 - b] = None
            inflight[1 - b] = gather(ch + 1, 1 - b)
        inflight[b].wait()
        pdesc.wait()

        def run_add(_b=b):
            @plsc.parallel_loop(0, CHUNK * VPR, unroll=32)
            def _(i):
                r = i >> 6
                c = (i & (VPR - 1)) * LANES
                sl = pl.ds(c, LANES)
                plsc.addupdate(tok_v.at[_b, r, sl], pos_v[r, sl])

        if ch == 0:
            run_add()
            @pl.when(wid == 0)
            def _():
                for k in range(VPR):
                    sl = pl.ds(k * LANES, LANES)
                    pos_v[0, sl] = tok_v[0, 0, sl]
                    tok_v[0, 0, sl] = stage_v[NSPECIAL - 1, sl]
                pltpu.async_copy(tok_v.at[0],
                                 out_hbm.at[pl.ds(NSPECIAL, CHUNK)], wsems[0])

                def shift_spec(i, carry):
                    r = NSPECIAL - 2 - i
                    for k in range(VPR):
                        sl = pl.ds(k * LANES, LANES)
                        stage_v[r + 1, sl] = stage_v[r, sl]
                    return carry
                lax.fori_loop(0, NSPECIAL - 1, shift_spec, 0)
                for k in range(VPR):
                    sl = pl.ds(k * LANES, LANES)
                    stage_v[0, sl] = pos_v[0, sl]
                pltpu.sync_copy(stage_v, out_hbm.at[pl.ds(0, NSPECIAL)])

            @pl.when(wid != 0)
            def _():
                pltpu.async_copy(tok_v.at[0],
                                 out_hbm.at[pl.ds(base + NSPECIAL, CHUNK)],
                                 wsems[0])
            writes[0] = pltpu.make_async_copy(
                tok_v.at[0], out_hbm.at[pl.ds(base + NSPECIAL, CHUNK)],
                wsems[0])
        else:
            run_add()
            writes[b] = pltpu.async_copy(
                tok_v.at[b], out_hbm.at[pl.ds(row0 + NSPECIAL, CHUNK)],
                wsems[b])
        if ch + 1 < NCHUNK:
            pdesc = issue_pos(ch + 1)

    for w in writes:
        if w is not None:
            w.wait()


_sc_kernel = functools.partial(
    pl.kernel,
    out_type=jax.ShapeDtypeStruct((LROWS, DIM), jnp.float32),
    mesh=plsc.VectorSubcoreMesh(core_axis_name="c", subcore_axis_name="s"),
    scratch_types=[
        pltpu.VMEM((RW,), jnp.int32),
        pltpu.VMEM((2, CHUNK, DIM), jnp.float32),
        pltpu.VMEM((CHUNK, DIM), jnp.float32),
        pltpu.VMEM((NSPECIAL, DIM), jnp.float32),
        pltpu.SemaphoreType.DMA,
        pltpu.SemaphoreType.DMA,
        pltpu.SemaphoreType.DMA,
        pltpu.SemaphoreType.DMA,
        pltpu.SemaphoreType.DMA,
    ],
)(_sc_body)


def kernel(input_ids, token_table, pos_table, special_token_embedding):
    ids = input_ids.reshape(LROWS)
    spec = special_token_embedding.reshape(NSPECIAL, DIM)
    out = _sc_kernel(ids, token_table, pos_table, spec)
    return out.reshape(1, LROWS, DIM)


# speedup vs baseline: 1.0085x; 1.0085x over previous
import functools

import jax
import jax.numpy as jnp
from jax import lax
from jax.experimental import pallas as pl
from jax.experimental.pallas import tpu as pltpu
from jax.experimental.pallas import tpu_sc as plsc

VOCAB = 49408
MAXPOS = 8192
DIM = 1024
NSPECIAL = 16
LROWS = MAXPOS + NSPECIAL

NC = 2
NS = 16
LANES = 16
NW = NC * NS
RW = MAXPOS // NW
CHUNK = 32
NCHUNK = RW // CHUNK
VPR = DIM // LANES


def _sc_body(ids_hbm, tok_hbm, pos_hbm, spec_hbm, out_hbm,
             idx_all, tok_v, pos_v, stage_v,
             gsem0, gsem1, psem, wsem0, wsem1):
    gsems = (gsem0, gsem1)
    wsems = (wsem0, wsem1)
    wid = lax.axis_index("s") * NC + lax.axis_index("c")
    base = wid * RW

    pltpu.sync_copy(ids_hbm.at[pl.ds(NSPECIAL + base, RW)], idx_all)

    def gather(ch, b):
        return pltpu.async_copy(
            tok_hbm.at[idx_all.at[pl.ds(ch * CHUNK, CHUNK)]], tok_v.at[b],
            gsems[b])

    def issue_pos(ch):
        return pltpu.async_copy(pos_hbm.at[pl.ds(base + ch * CHUNK, CHUNK)],
                                pos_v, psem)

    inflight = [None, None]
    writes = [None, None]
    inflight[0] = gather(0, 0)
    pdesc = issue_pos(0)

    @pl.when(wid == 0)
    def _():
        pltpu.sync_copy(spec_hbm, stage_v)

    for ch in range(NCHUNK):
        b = ch & 1
        row0 = base + ch * CHUNK
        if ch + 1 < NCHUNK:
            if writes[1 - b] is not None:
                for _d in (writes[1 - b] if isinstance(writes[1 - b], tuple)
                           else (writes[1 - b],)):
                    _d.wait()
                writes[1 - b] = None
            inflight[1 - b] = gather(ch + 1, 1 - b)
        inflight[b].wait()
        pdesc.wait()

        def run_add(lo=0, hi=CHUNK, _b=b):
            @plsc.parallel_loop(lo * VPR, hi * VPR, unroll=16)
            def _(i):
                r = i >> 6
                c = (i & (VPR - 1)) * LANES
                sl = pl.ds(c, LANES)
                plsc.addupdate(tok_v.at[_b, r, sl], pos_v[r, sl])

        if ch == 0:
            run_add()
            @pl.when(wid == 0)
            def _():
                for k in range(VPR):
                    sl = pl.ds(k * LANES, LANES)
                    pos_v[0, sl] = tok_v[0, 0, sl]
                    tok_v[0, 0, sl] = stage_v[NSPECIAL - 1, sl]
                pltpu.async_copy(tok_v.at[0],
                                 out_hbm.at[pl.ds(NSPECIAL, CHUNK)], wsems[0])

                def shift_spec(i, carry):
                    r = NSPECIAL - 2 - i
                    for k in range(VPR):
                        sl = pl.ds(k * LANES, LANES)
                        stage_v[r + 1, sl] = stage_v[r, sl]
                    return carry
                lax.fori_loop(0, NSPECIAL - 1, shift_spec, 0)
                for k in range(VPR):
                    sl = pl.ds(k * LANES, LANES)
                    stage_v[0, sl] = pos_v[0, sl]
                pltpu.sync_copy(stage_v, out_hbm.at[pl.ds(0, NSPECIAL)])

            @pl.when(wid != 0)
            def _():
                pltpu.async_copy(tok_v.at[0],
                                 out_hbm.at[pl.ds(base + NSPECIAL, CHUNK)],
                                 wsems[0])
            writes[0] = pltpu.make_async_copy(
                tok_v.at[0], out_hbm.at[pl.ds(base + NSPECIAL, CHUNK)],
                wsems[0])
        else:
            half = CHUNK // 2
            run_add(0, half)
            d1 = pltpu.async_copy(
                tok_v.at[b].at[pl.ds(0, half)],
                out_hbm.at[pl.ds(row0 + NSPECIAL, half)], wsems[b])
            run_add(half, CHUNK)
            d2 = pltpu.async_copy(
                tok_v.at[b].at[pl.ds(half, half)],
                out_hbm.at[pl.ds(row0 + NSPECIAL + half, half)], wsems[b])
            writes[b] = (d1, d2)
        if ch + 1 < NCHUNK:
            pdesc = issue_pos(ch + 1)

    for w in writes:
        if w is not None:
            for _d in (w if isinstance(w, tuple) else (w,)):
                _d.wait()


_sc_kernel = functools.partial(
    pl.kernel,
    out_type=jax.ShapeDtypeStruct((LROWS, DIM), jnp.float32),
    mesh=plsc.VectorSubcoreMesh(core_axis_name="c", subcore_axis_name="s"),
    scratch_types=[
        pltpu.VMEM((RW,), jnp.int32),
        pltpu.VMEM((2, CHUNK, DIM), jnp.float32),
        pltpu.VMEM((CHUNK, DIM), jnp.float32),
        pltpu.VMEM((NSPECIAL, DIM), jnp.float32),
        pltpu.SemaphoreType.DMA,
        pltpu.SemaphoreType.DMA,
        pltpu.SemaphoreType.DMA,
        pltpu.SemaphoreType.DMA,
        pltpu.SemaphoreType.DMA,
    ],
)(_sc_body)


def kernel(input_ids, token_table, pos_table, special_token_embedding):
    ids = input_ids.reshape(LROWS)
    spec = special_token_embedding.reshape(NSPECIAL, DIM)
    out = _sc_kernel(ids, token_table, pos_table, spec)
    return out.reshape(1, LROWS, DIM)


# speedup vs baseline: 1.0559x; 1.0470x over previous
import functools

import jax
import jax.numpy as jnp
from jax import lax
from jax.experimental import pallas as pl
from jax.experimental.pallas import tpu as pltpu
from jax.experimental.pallas import tpu_sc as plsc

VOCAB = 49408
MAXPOS = 8192
DIM = 1024
NSPECIAL = 16
LROWS = MAXPOS + NSPECIAL

NC = 2
NS = 16
LANES = 16
NW = NC * NS
RW = MAXPOS // NW
CHUNK = 32
NCHUNK = RW // CHUNK
VPR = DIM // LANES


def _sc_body(ids_hbm, tok_hbm, pos_hbm, spec_hbm, out_hbm,
             idx_all, tok_v, pos_v, stage_v,
             gsem0, gsem1, psem, wsem0, wsem1):
    gsems = (gsem0, gsem1)
    wsems = (wsem0, wsem1)
    wid = lax.axis_index("s") * NC + lax.axis_index("c")
    base = wid * RW

    pltpu.sync_copy(ids_hbm.at[pl.ds(NSPECIAL + base, RW)], idx_all)

    def gather(ch, b):
        return pltpu.async_copy(
            tok_hbm.at[idx_all.at[pl.ds(ch * CHUNK, CHUNK)]], tok_v.at[b],
            gsems[b])

    def issue_pos(ch):
        return pltpu.async_copy(pos_hbm.at[pl.ds(base + ch * CHUNK, CHUNK)],
                                pos_v, psem)

    inflight = [None, None]
    writes = [None, None]
    inflight[0] = gather(0, 0)
    pdesc = issue_pos(0)

    @pl.when(wid == 0)
    def _():
        pltpu.sync_copy(spec_hbm, stage_v)

    for ch in range(NCHUNK):
        b = ch & 1
        row0 = base + ch * CHUNK
        if ch + 1 < NCHUNK:
            if writes[1 - b] is not None:
                writes[1 - b].wait()
                writes[1 - b] = None
            inflight[1 - b] = gather(ch + 1, 1 - b)
        inflight[b].wait()
        pdesc.wait()

        def run_add(_b=b):
            @plsc.parallel_loop(0, CHUNK * VPR, unroll=16)
            def _(i):
                r = i >> 6
                c = (i & (VPR - 1)) * LANES
                sl = pl.ds(c, LANES)
                plsc.addupdate(tok_v.at[_b, r, sl], pos_v[r, sl])

        if ch == 0:
            run_add()
            @pl.when(wid == 0)
            def _():
                for k in range(VPR):
                    sl = pl.ds(k * LANES, LANES)
                    pos_v[0, sl] = tok_v[0, 0, sl]
                    tok_v[0, 0, sl] = stage_v[NSPECIAL - 1, sl]
                pltpu.async_copy(tok_v.at[0],
                                 out_hbm.at[pl.ds(NSPECIAL, CHUNK)], wsems[0])

                def shift_spec(i, carry):
                    r = NSPECIAL - 2 - i
                    for k in range(VPR):
                        sl = pl.ds(k * LANES, LANES)
                        stage_v[r + 1, sl] = stage_v[r, sl]
                    return carry
                lax.fori_loop(0, NSPECIAL - 1, shift_spec, 0)
                for k in range(VPR):
                    sl = pl.ds(k * LANES, LANES)
                    stage_v[0, sl] = pos_v[0, sl]
                pltpu.sync_copy(stage_v, out_hbm.at[pl.ds(0, NSPECIAL)])

            @pl.when(wid != 0)
            def _():
                pltpu.async_copy(tok_v.at[0],
                                 out_hbm.at[pl.ds(base + NSPECIAL, CHUNK)],
                                 wsems[0])
            writes[0] = pltpu.make_async_copy(
                tok_v.at[0], out_hbm.at[pl.ds(base + NSPECIAL, CHUNK)],
                wsems[0])
        else:
            run_add()
            writes[b] = pltpu.async_copy(
                tok_v.at[b], out_hbm.at[pl.ds(row0 + NSPECIAL, CHUNK)],
                wsems[b])
        if ch + 1 < NCHUNK:
            pdesc = issue_pos(ch + 1)

    for w in writes:
        if w is not None:
            w.wait()


_sc_kernel = functools.partial(
    pl.kernel,
    out_type=jax.ShapeDtypeStruct((LROWS, DIM), jnp.float32),
    mesh=plsc.VectorSubcoreMesh(core_axis_name="c", subcore_axis_name="s"),
    scratch_types=[
        pltpu.VMEM((RW,), jnp.int32),
        pltpu.VMEM((2, CHUNK, DIM), jnp.float32),
        pltpu.VMEM((CHUNK, DIM), jnp.float32),
        pltpu.VMEM((NSPECIAL, DIM), jnp.float32),
        pltpu.SemaphoreType.DMA,
        pltpu.SemaphoreType.DMA,
        pltpu.SemaphoreType.DMA,
        pltpu.SemaphoreType.DMA,
        pltpu.SemaphoreType.DMA,
    ],
)(_sc_body)


def kernel(input_ids, token_table, pos_table, special_token_embedding):
    ids = input_ids.reshape(LROWS)
    spec = special_token_embedding.reshape(NSPECIAL, DIM)
    out = _sc_kernel(ids, token_table, pos_table, spec)
    return out.reshape(1, LROWS, DIM)
